# baseline (speedup 1.0000x reference)
import functools

import jax
import jax.numpy as jnp
from jax import lax
from jax.experimental import pallas as pl
from jax.experimental.pallas import tpu as pltpu
from jax.experimental.pallas import tpu_sc as plsc

D_MODEL = 1024
D_FF = 1024
N_EXP = 8
T = 2048
K = 2
P = K * T
TILE = 256
P_MAX = P + N_EXP * TILE
N_TILES = P_MAX // TILE

NC = 2
NS = 16
NW = NC * NS
PAIRS_PER_W = P // NW
HALF_W = PAIRS_PER_W // 2
CHUNKS_ALL = P // 16
CHUNKS_MINE = PAIRS_PER_W // 16
TOK_PER_W = T // NW


def _gate_kernel(x_ref, wg_ref, e1_ref, e2_ref, w1_ref, w2_ref):
    logits = jnp.dot(x_ref[...], wg_ref[...], preferred_element_type=jnp.float32)
    eids = lax.broadcasted_iota(jnp.int32, (T, N_EXP), 1)
    m1 = jnp.max(logits, axis=1, keepdims=True)
    e1 = jnp.min(jnp.where(logits == m1, eids, N_EXP), axis=1, keepdims=True)
    masked = jnp.where(eids == e1, -jnp.inf, logits)
    m2 = jnp.max(masked, axis=1, keepdims=True)
    e2 = jnp.min(jnp.where(masked == m2, eids, N_EXP), axis=1, keepdims=True)
    w1 = 1.0 / (1.0 + jnp.exp(m2 - m1))
    e1_ref[...] = e1
    e2_ref[...] = e2
    w1_ref[...] = w1
    w2_ref[...] = 1.0 - w1


def _route_kernel(pe_hbm, wv_hbm, x_hbm,
                  xs_hbm, ws_hbm, slots_hbm, tmap_hbm, tval_hbm,
                  pe_v, wt_v, sla_v, slb_v, wrow_v, xrow_v, tm_v, tv_v,
                  sem0, sem1, sem2):
    wid = lax.axis_index("s") * NC + lax.axis_index("c")
    ones16 = jnp.ones((16,), jnp.int32)
    zeros16 = jnp.zeros((16,), jnp.int32)
    lanes16 = lax.iota(jnp.int32, 16)

    gdn = lax.GatherDimensionNumbers(
        offset_dims=(), collapsed_slice_dims=(0,), start_index_map=(0,))

    def _gather16(x, idx):
        return lax.gather(x, idx[:, None], gdn, (1,),
                          mode=lax.GatherScatterMode.PROMISE_IN_BOUNDS)

    def _prefix(m):
        y = m
        for k in (1, 2, 4, 8):
            g = _gather16(y, jnp.maximum(lanes16 - k, zeros16))
            y = y + jnp.where(lanes16 >= k, g, zeros16)
        return y

    def _splat_last(y):
        return _gather16(y, jnp.full((16,), 15, jnp.int32))

    t0 = (wid % NS) * PAIRS_PER_W
    xcopy = pltpu.async_copy(x_hbm.at[pl.ds(t0, HALF_W)], xrow_v, sem0)

    pltpu.sync_copy(pe_hbm, pe_v)
    pltpu.sync_copy(wv_hbm.at[pl.ds(wid * PAIRS_PER_W, PAIRS_PER_W)], wt_v)

    def hist_body(i, carry):
        tot, pref = carry
        sel = jnp.where(i < wid * CHUNKS_MINE, 1, 0)
        v = pe_v[pl.ds(i * 16, 16)]
        tot_new = []
        pref_new = []
        for e in range(N_EXP):
            m = jnp.where(v == e, ones16, zeros16)
            tot_new.append(tot[e] + m)
            pref_new.append(pref[e] + m * sel)
        return tot_new, pref_new

    tot0 = [zeros16] * N_EXP
    pref0 = [zeros16] * N_EXP
    tot_acc, pref_acc = lax.fori_loop(0, CHUNKS_ALL, hist_body, (tot0, pref0))

    tot = [_splat_last(_prefix(tot_acc[e])) for e in range(N_EXP)]
    pref = [_splat_last(_prefix(pref_acc[e])) for e in range(N_EXP)]
    padded = [((tot[e] + (TILE - 1)) >> 8) << 8 for e in range(N_EXP)]
    start = []
    s = zeros16
    for e in range(N_EXP):
        start.append(s)
        s = s + padded[e]
    total_padded = s

    base = [start[e] + pref[e] for e in range(N_EXP)]
    for c in range(CHUNKS_MINE):
        v = pe_v[pl.ds(wid * PAIRS_PER_W + c * 16, 16)]
        slot = zeros16
        for e in range(N_EXP):
            m = jnp.where(v == e, ones16, zeros16)
            incl = _prefix(m)
            slot = slot + m * (base[e] + incl - m)
            base[e] = base[e] + _splat_last(incl)
        if c < CHUNKS_MINE // 2:
            sla_v[pl.ds(c * 16, 16)] = slot
        else:
            slb_v[pl.ds((c - CHUNKS_MINE // 2) * 16, 16)] = slot

    fzeros16 = jnp.zeros((16,), jnp.float32)
    for c in range(CHUNKS_MINE):
        wtc = wt_v[pl.ds(c * 16, 16)]
        for j in range(16):
            wrow_v[c * 16 + j, pl.ds(0, 16)] = jnp.where(
                lanes16 == j, wtc, fzeros16)

    pltpu.sync_copy(sla_v, slots_hbm.at[pl.ds(wid * PAIRS_PER_W, HALF_W)])
    pltpu.sync_copy(slb_v, slots_hbm.at[pl.ds(wid * PAIRS_PER_W + HALF_W, HALF_W)])

    xcopy.wait()
    pltpu.sync_copy(xrow_v, xs_hbm.at[sla_v])
    xcopy2 = pltpu.async_copy(x_hbm.at[pl.ds(t0 + HALF_W, HALF_W)], xrow_v, sem1)
    pltpu.sync_copy(wrow_v.at[pl.ds(0, HALF_W)], ws_hbm.at[sla_v])
    pltpu.sync_copy(wrow_v.at[pl.ds(HALF_W, HALF_W)], ws_hbm.at[slb_v])

    @pl.when(wid == 0)
    def _():
        for c2 in range(2):
            pos = (lanes16 + c2 * 16) * TILE
            texp = zeros16
            for e in range(N_EXP):
                inr = (pos >= start[e]) & (pos < start[e] + padded[e])
                texp = texp + jnp.where(inr, ones16, zeros16) * e
            active = jnp.where(pos < total_padded, ones16, zeros16)
            tm_v[pl.ds(c2 * 16, 16)] = active * texp + (1 - active) * (N_EXP - 1)
            tv_v[pl.ds(c2 * 16, 16)] = active
        pltpu.sync_copy(tm_v, tmap_hbm)
        pltpu.sync_copy(tv_v, tval_hbm)

    xcopy2.wait()
    pltpu.sync_copy(xrow_v, xs_hbm.at[slb_v])


def _gmm_kernel(tmap_ref, tval_ref, xs_ref, ws_ref, w1_ref, b1_ref,
                w2_ref, b2_ref, y_ref):
    j = pl.program_id(0)

    @pl.when(tval_ref[j] == 1)
    def _():
        h = jnp.dot(xs_ref[...].astype(jnp.bfloat16),
                    w1_ref[0].astype(jnp.bfloat16),
                    preferred_element_type=jnp.float32)
        h = h + b1_ref[0]
        h = h * jax.nn.sigmoid(h)
        y = jnp.dot(h.astype(jnp.bfloat16), w2_ref[0].astype(jnp.bfloat16),
                    preferred_element_type=jnp.float32)
        y = y + b2_ref[0]
        w_row = jnp.sum(ws_ref[...][:, 0:16], axis=1, keepdims=True)
        y_ref[...] = y * w_row


def _combine_kernel(y_hbm, slots_hbm, out_hbm, sl1_v, sl2_v, a1_v, a2_v,
                    sem0, sem1):
    wid = lax.axis_index("s") * NC + lax.axis_index("c")
    tb = wid * TOK_PER_W
    pltpu.sync_copy(slots_hbm.at[pl.ds(tb, TOK_PER_W)], sl1_v)
    pltpu.sync_copy(slots_hbm.at[pl.ds(T + tb, TOK_PER_W)], sl2_v)

    def body(q, carry):
        c1 = pltpu.async_copy(y_hbm.at[sl1_v.at[pl.ds(q * 16, 16)]], a1_v, sem0)
        c2 = pltpu.async_copy(y_hbm.at[sl2_v.at[pl.ds(q * 16, 16)]], a2_v, sem1)
        c1.wait()
        c2.wait()
        for r in range(16):
            for c in range(D_MODEL // 16):
                a1_v[r, pl.ds(c * 16, 16)] = (
                    a1_v[r, pl.ds(c * 16, 16)] + a2_v[r, pl.ds(c * 16, 16)])
        pltpu.sync_copy(a1_v, out_hbm.at[pl.ds(tb + q * 16, 16)])
        return carry

    lax.fori_loop(0, TOK_PER_W // 16, body, 0)


_SC_MESH = plsc.VectorSubcoreMesh(core_axis_name="c", subcore_axis_name="s")


@jax.jit
def kernel(x, Wg, W1, b1, W2, b2):
    e1, e2, w1, w2 = pl.pallas_call(
        _gate_kernel,
        out_shape=(
            jax.ShapeDtypeStruct((T, 1), jnp.int32),
            jax.ShapeDtypeStruct((T, 1), jnp.int32),
            jax.ShapeDtypeStruct((T, 1), jnp.float32),
            jax.ShapeDtypeStruct((T, 1), jnp.float32),
        ),
    )(x, Wg)

    pe = jnp.concatenate([e1.reshape(T), e2.reshape(T)])
    wv = jnp.concatenate([w1.reshape(T), w2.reshape(T)])

    route = functools.partial(
        pl.kernel,
        out_type=(
            jax.ShapeDtypeStruct((P_MAX, D_MODEL), jnp.float32),
            jax.ShapeDtypeStruct((P_MAX, 128), jnp.float32),
            jax.ShapeDtypeStruct((P,), jnp.int32),
            jax.ShapeDtypeStruct((32,), jnp.int32),
            jax.ShapeDtypeStruct((32,), jnp.int32),
        ),
        mesh=_SC_MESH,
        scratch_types=[
            pltpu.VMEM((P,), jnp.int32),
            pltpu.VMEM((PAIRS_PER_W,), jnp.float32),
            pltpu.VMEM((HALF_W,), jnp.int32),
            pltpu.VMEM((HALF_W,), jnp.int32),
            pltpu.VMEM((PAIRS_PER_W, 128), jnp.float32),
            pltpu.VMEM((HALF_W, D_MODEL), jnp.float32),
            pltpu.VMEM((32,), jnp.int32),
            pltpu.VMEM((32,), jnp.int32),
            pltpu.SemaphoreType.DMA,
            pltpu.SemaphoreType.DMA,
            pltpu.SemaphoreType.DMA,
        ],
    )(_route_kernel)
    xs, ws, slots, tmap, tval = route(pe, wv, x)

    y_s = pl.pallas_call(
        _gmm_kernel,
        grid_spec=pltpu.PrefetchScalarGridSpec(
            num_scalar_prefetch=2,
            grid=(N_TILES,),
            in_specs=[
                pl.BlockSpec((TILE, D_MODEL), lambda j, tm, tv: (j, 0)),
                pl.BlockSpec((TILE, 128), lambda j, tm, tv: (j, 0)),
                pl.BlockSpec((1, D_MODEL, D_FF), lambda j, tm, tv: (tm[j], 0, 0)),
                pl.BlockSpec((1, 1, D_FF), lambda j, tm, tv: (tm[j], 0, 0)),
                pl.BlockSpec((1, D_FF, D_MODEL), lambda j, tm, tv: (tm[j], 0, 0)),
                pl.BlockSpec((1, 1, D_MODEL), lambda j, tm, tv: (tm[j], 0, 0)),
            ],
            out_specs=pl.BlockSpec((TILE, D_MODEL), lambda j, tm, tv: (j, 0)),
        ),
        out_shape=jax.ShapeDtypeStruct((P_MAX, D_MODEL), jnp.float32),
    )(tmap, tval, xs, ws, W1, b1.reshape(N_EXP, 1, D_FF), W2,
      b2.reshape(N_EXP, 1, D_MODEL))

    combine = functools.partial(
        pl.kernel,
        out_type=jax.ShapeDtypeStruct((T, D_MODEL), jnp.float32),
        mesh=_SC_MESH,
        scratch_types=[
            pltpu.VMEM((TOK_PER_W,), jnp.int32),
            pltpu.VMEM((TOK_PER_W,), jnp.int32),
            pltpu.VMEM((16, D_MODEL), jnp.float32),
            pltpu.VMEM((16, D_MODEL), jnp.float32),
            pltpu.SemaphoreType.DMA,
            pltpu.SemaphoreType.DMA,
        ],
    )(_combine_kernel)
    out = combine(y_s, slots)
    return out
